# baseline (speedup 1.0000x reference)
import jax
import jax.numpy as jnp
from jax import lax
from jax.experimental import pallas as pl
from jax.experimental.pallas import tpu as pltpu
from jax.experimental.pallas import tpu_sc as plsc

VOCAB = 100000
EMBED = 64
BATCH = 16384
LANES = 16
NC = 2
NS = 16
W = 128
ROWS = BATCH // W
RPW = ROWS // NS
VH = 49920
TAIL0 = 2 * VH
TAIL = VOCAB - TAIL0
ICH = 8192
IROWS = ICH // W
PAIRS_PER_W = EMBED // (NC * NS)
NSTREAM = PAIRS_PER_W * 2 * 2


def _glove_body(wu_hbm, wv_hbm, ut_hbm, ub_hbm, vt_hbm, vb_hbm,
                tu_hbm, tv_hbm, out0_hbm, out1_hbm, vec0, vec1, g1, idxb, tailb,
                sbuf, bbuf, bidx, ridx, shared, sidx_u, sidx_v, sem, sem2):
    c = lax.axis_index("c")
    s = lax.axis_index("s")
    lane = lax.iota(jnp.int32, LANES)
    zero16 = jnp.zeros((LANES,), jnp.float32)
    fs = [c * (NS * PAIRS_PER_W) + s * PAIRS_PER_W + k
          for k in range(PAIRS_PER_W)]

    for i in range(W // LANES):
        ridx[pl.ds(i * LANES, LANES)] = lane + i * LANES

    @pl.when(s == 0)
    def _():
        pltpu.sync_copy(wu_hbm, sidx_u)
        pltpu.sync_copy(wv_hbm, sidx_v)

    for r in range(RPW):
        for q in range(W // LANES):
            sbuf[r, pl.ds(q * LANES, LANES)] = zero16
    my_rows = pl.ds(s * RPW, RPW)
    pltpu.sync_copy(sbuf, shared.at[my_rows])
    plsc.subcore_barrier()

    def mk(t):
        k, ph, h = t // 4, (t // 2) % 2, t % 2
        tab = vt_hbm if ph else ut_hbm
        return pltpu.make_async_copy(
            tab.at[:, pl.ds(h * VH, VH)].at[fs[k]],
            vec0 if t % 2 == 0 else vec1, sem)

    def scan(ph, h, ch, buf):
        lo = h * VH

        @plsc.parallel_loop(0, IROWS, unroll=2)
        def body(r2):
            r = ch * IROWS + r2
            for qq in range(W // LANES):
                sl = pl.ds(r2 * W + qq * LANES, LANES)
                csl = pl.ds(qq * LANES, LANES)
                iu = idxb[sl]
                li = iu - lo
                m = (li >= 0) & (li < VH)
                g = plsc.load_gather(buf, [jnp.where(m, li, 0)])
                if h == 1:
                    lt = iu - TAIL0
                    mt = lt >= 0
                    gt = plsc.load_gather(tailb, [jnp.where(mt, lt, 0)])
                    g = jnp.where(mt, gt, g)
                    m = m | mt
                if ph == 0:
                    if h == 0:
                        g1[r, csl] = jnp.where(m, g, 0.0)
                    else:
                        g1[r, csl] = g1[r, csl] + jnp.where(m, g, 0.0)
                else:
                    g1[r, csl] = g1[r, csl] * jnp.where(m, g, 1.0)

    mk(0).start()
    mk(1).start()
    for t in range(NSTREAM):
        k, ph, h = t // 4, (t // 2) % 2, t % 2
        if h == 0:
            ttab = tv_hbm if ph else tu_hbm
            pltpu.sync_copy(ttab.at[fs[k]], tailb)
        mk(t).wait()
        idx_sp = sidx_v if ph else sidx_u
        for ch in range(BATCH // ICH):
            pltpu.sync_copy(idx_sp.at[pl.ds(ch * ICH, ICH)], idxb)
            scan(ph, h, ch, vec0 if t % 2 == 0 else vec1)
        if t + 2 < NSTREAM:
            mk(t + 2).start()
        if t % 4 == 3:
            pltpu.sync_copy(g1, shared.at[ridx], add=True)

    plsc.subcore_barrier()

    pltpu.sync_copy(shared.at[my_rows], sbuf)

    def add_bias(bias_hbm, widx_hbm):
        half_rows = RPW // 2
        for hh in range(2):
            base = (s * RPW + hh * half_rows) * W
            pltpu.sync_copy(widx_hbm.at[pl.ds(base, half_rows * W)], bidx)
            copies = [pltpu.make_async_copy(
                bias_hbm.at[bidx.at[pl.ds(j * W, W)]],
                bbuf.at[pl.ds(j * W, W)], sem2)
                for j in range(half_rows)]
            for cp in copies:
                cp.start()
            for cp in copies:
                cp.wait()
            for r in range(half_rows):
                for q in range(W // LANES):
                    sl = pl.ds(q * LANES, LANES)
                    sbuf[hh * half_rows + r, sl] = (
                        sbuf[hh * half_rows + r, sl]
                        + bbuf[pl.ds(r * W + q * LANES, LANES)])

    @pl.when(c == 0)
    def _():
        add_bias(ub_hbm, wu_hbm)
        pltpu.sync_copy(sbuf, out0_hbm.at[my_rows])

    @pl.when(c == 1)
    def _():
        add_bias(vb_hbm, wv_hbm)
        pltpu.sync_copy(sbuf, out1_hbm.at[my_rows])


def _glove_sc(wu, wv, ut, ub1, vt, vb1, tu, tv):
    mesh = plsc.VectorSubcoreMesh(core_axis_name="c", subcore_axis_name="s")
    f = pl.kernel(
        _glove_body,
        out_type=(jax.ShapeDtypeStruct((ROWS, W), jnp.float32),
                  jax.ShapeDtypeStruct((ROWS, W), jnp.float32)),
        mesh=mesh,
        scratch_types=[
            pltpu.VMEM((VH,), jnp.float32),
            pltpu.VMEM((VH,), jnp.float32),
            pltpu.VMEM((ROWS, W), jnp.float32),
            pltpu.VMEM((ICH,), jnp.int32),
            pltpu.VMEM((TAIL,), jnp.float32),
            pltpu.VMEM((RPW, W), jnp.float32),
            pltpu.VMEM((RPW * W // 2,), jnp.float32),
            pltpu.VMEM((RPW * W // 2,), jnp.int32),
            pltpu.VMEM((W,), jnp.int32),
            pltpu.VMEM_SHARED((ROWS, W), jnp.float32),
            pltpu.VMEM_SHARED((BATCH,), jnp.int32),
            pltpu.VMEM_SHARED((BATCH,), jnp.int32),
            pltpu.SemaphoreType.DMA,
            pltpu.SemaphoreType.DMA,
        ],
        compiler_params=pltpu.CompilerParams(needs_layout_passes=False),
    )
    return f(wu, wv, ut, ub1, vt, vb1, tu, tv)


def kernel(word_u, word_v, in_embed, in_bias, out_embed, out_bias):
    wu = word_u.astype(jnp.int32)
    wv = word_v.astype(jnp.int32)
    ut = in_embed.T
    vt = out_embed.T
    out0, out1 = _glove_sc(wu, wv, ut, in_bias.reshape(VOCAB),
                           vt, out_bias.reshape(VOCAB),
                           ut[:, TAIL0:], vt[:, TAIL0:])
    return (out0 + out1).reshape(BATCH)


# speedup vs baseline: 1.0960x; 1.0960x over previous
import jax
import jax.numpy as jnp
from jax import lax
from jax.experimental import pallas as pl
from jax.experimental.pallas import tpu as pltpu
from jax.experimental.pallas import tpu_sc as plsc

VOCAB = 100000
EMBED = 64
BATCH = 16384
LANES = 16
NC = 2
NS = 16
W = 128
ROWS = BATCH // W
RPW = ROWS // NS
VH = 49920
TAIL0 = 2 * VH
TAIL = VOCAB - TAIL0
ICH = 8192
IROWS = ICH // W
PAIRS_PER_W = EMBED // (NC * NS)
NSTREAM = PAIRS_PER_W * 2 * 2


def _glove_body(wu_hbm, wv_hbm, ut_hbm, ub_hbm, vt_hbm, vb_hbm,
                tu_hbm, tv_hbm, out0_hbm, out1_hbm, vec0, vec1, g1, idxb, tailb,
                sbuf, bbuf, bidx, ridx, shared, sidx_u, sidx_v, sem, sem2):
    c = lax.axis_index("c")
    s = lax.axis_index("s")
    lane = lax.iota(jnp.int32, LANES)
    zero16 = jnp.zeros((LANES,), jnp.float32)
    fs = [c * (NS * PAIRS_PER_W) + s * PAIRS_PER_W + k
          for k in range(PAIRS_PER_W)]

    for i in range(W // LANES):
        ridx[pl.ds(i * LANES, LANES)] = lane + i * LANES

    @pl.when(s == 0)
    def _():
        pltpu.sync_copy(wu_hbm, sidx_u)
        pltpu.sync_copy(wv_hbm, sidx_v)

    for r in range(RPW):
        for q in range(W // LANES):
            sbuf[r, pl.ds(q * LANES, LANES)] = zero16
    my_rows = pl.ds(s * RPW, RPW)
    pltpu.sync_copy(sbuf, shared.at[my_rows])
    plsc.subcore_barrier()

    def mk(t):
        k, ph, h = t // 4, (t // 2) % 2, t % 2
        tab = vt_hbm if ph else ut_hbm
        return pltpu.make_async_copy(
            tab.at[:, pl.ds(h * VH, VH)].at[fs[k]],
            (vec0 if t % 2 == 0 else vec1).at[pl.ds(0, VH)], sem)

    def scan(ph, h, ch, buf):
        lo = h * VH
        hi = VH if h == 0 else VH + TAIL

        @plsc.parallel_loop(0, IROWS, unroll=1)
        def body(r2):
            r = ch * IROWS + r2
            for qq in range(W // LANES):
                sl = pl.ds(r2 * W + qq * LANES, LANES)
                csl = pl.ds(qq * LANES, LANES)
                li = idxb[sl] - lo
                m = (li >= 0) & (li < hi)
                g = plsc.load_gather(buf, [jnp.where(m, li, 0)])
                if ph == 0:
                    if h == 0:
                        g1[r, csl] = jnp.where(m, g, 0.0)
                    else:
                        g1[r, csl] = g1[r, csl] + jnp.where(m, g, 0.0)
                else:
                    g1[r, csl] = g1[r, csl] * jnp.where(m, g, 1.0)

    mk(0).start()
    mk(1).start()
    for t in range(NSTREAM):
        k, ph, h = t // 4, (t // 2) % 2, t % 2
        mk(t).wait()
        if h == 1:
            ttab = tv_hbm if ph else tu_hbm
            buf = vec0 if t % 2 == 0 else vec1
            pltpu.sync_copy(ttab.at[fs[k]], tailb)
            for i in range(TAIL // LANES):
                buf[pl.ds(VH + i * LANES, LANES)] = tailb[pl.ds(i * LANES,
                                                               LANES)]
        idx_sp = sidx_v if ph else sidx_u
        for ch in range(BATCH // ICH):
            pltpu.sync_copy(idx_sp.at[pl.ds(ch * ICH, ICH)], idxb)
            scan(ph, h, ch, vec0 if t % 2 == 0 else vec1)
        if t + 2 < NSTREAM:
            mk(t + 2).start()
        if t % 4 == 3:
            pltpu.sync_copy(g1, shared.at[ridx], add=True)

    plsc.subcore_barrier()

    pltpu.sync_copy(shared.at[my_rows], sbuf)

    def add_bias(bias_hbm, widx_hbm):
        half_rows = RPW // 2
        for hh in range(2):
            base = (s * RPW + hh * half_rows) * W
            pltpu.sync_copy(widx_hbm.at[pl.ds(base, half_rows * W)], bidx)
            copies = [pltpu.make_async_copy(
                bias_hbm.at[bidx.at[pl.ds(j * W, W)]],
                bbuf.at[pl.ds(j * W, W)], sem2)
                for j in range(half_rows)]
            for cp in copies:
                cp.start()
            for cp in copies:
                cp.wait()
            for r in range(half_rows):
                for q in range(W // LANES):
                    sl = pl.ds(q * LANES, LANES)
                    sbuf[hh * half_rows + r, sl] = (
                        sbuf[hh * half_rows + r, sl]
                        + bbuf[pl.ds(r * W + q * LANES, LANES)])

    @pl.when(c == 0)
    def _():
        add_bias(ub_hbm, wu_hbm)
        pltpu.sync_copy(sbuf, out0_hbm.at[my_rows])

    @pl.when(c == 1)
    def _():
        add_bias(vb_hbm, wv_hbm)
        pltpu.sync_copy(sbuf, out1_hbm.at[my_rows])


def _glove_sc(wu, wv, ut, ub1, vt, vb1, tu, tv):
    mesh = plsc.VectorSubcoreMesh(core_axis_name="c", subcore_axis_name="s")
    f = pl.kernel(
        _glove_body,
        out_type=(jax.ShapeDtypeStruct((ROWS, W), jnp.float32),
                  jax.ShapeDtypeStruct((ROWS, W), jnp.float32)),
        mesh=mesh,
        scratch_types=[
            pltpu.VMEM((VH + TAIL,), jnp.float32),
            pltpu.VMEM((VH + TAIL,), jnp.float32),
            pltpu.VMEM((ROWS, W), jnp.float32),
            pltpu.VMEM((ICH,), jnp.int32),
            pltpu.VMEM((TAIL,), jnp.float32),
            pltpu.VMEM((RPW, W), jnp.float32),
            pltpu.VMEM((RPW * W // 2,), jnp.float32),
            pltpu.VMEM((RPW * W // 2,), jnp.int32),
            pltpu.VMEM((W,), jnp.int32),
            pltpu.VMEM_SHARED((ROWS, W), jnp.float32),
            pltpu.VMEM_SHARED((BATCH,), jnp.int32),
            pltpu.VMEM_SHARED((BATCH,), jnp.int32),
            pltpu.SemaphoreType.DMA,
            pltpu.SemaphoreType.DMA,
        ],
        compiler_params=pltpu.CompilerParams(needs_layout_passes=False),
    )
    return f(wu, wv, ut, ub1, vt, vb1, tu, tv)


def kernel(word_u, word_v, in_embed, in_bias, out_embed, out_bias):
    wu = word_u.astype(jnp.int32)
    wv = word_v.astype(jnp.int32)
    ut = in_embed.T
    vt = out_embed.T
    out0, out1 = _glove_sc(wu, wv, ut, in_bias.reshape(VOCAB),
                           vt, out_bias.reshape(VOCAB),
                           ut[:, TAIL0:], vt[:, TAIL0:])
    return (out0 + out1).reshape(BATCH)


# speedup vs baseline: 1.1263x; 1.0276x over previous
import jax
import jax.numpy as jnp
from jax import lax
from jax.experimental import pallas as pl
from jax.experimental.pallas import tpu as pltpu
from jax.experimental.pallas import tpu_sc as plsc

VOCAB = 100000
EMBED = 64
BATCH = 16384
LANES = 16
NC = 2
NS = 16
W = 128
ROWS = BATCH // W
RPW = ROWS // NS
VH = 49920
TAIL0 = 2 * VH
TAIL = VOCAB - TAIL0
ICH = 8192
IROWS = ICH // W
PAIRS_PER_W = EMBED // (NC * NS)
NSTREAM = PAIRS_PER_W * 2 * 2


def _glove_body(wu_hbm, wv_hbm, ut_hbm, ub_hbm, vt_hbm, vb_hbm,
                tu_hbm, tv_hbm, out0_hbm, out1_hbm, vec0, vec1, g1, idxb, tailb,
                sbuf, bbuf, bidx, ridx, shared, sidx_u, sidx_v, sem, sem2):
    c = lax.axis_index("c")
    s = lax.axis_index("s")
    lane = lax.iota(jnp.int32, LANES)
    zero16 = jnp.zeros((LANES,), jnp.float32)
    fs = [c * (NS * PAIRS_PER_W) + s * PAIRS_PER_W + k
          for k in range(PAIRS_PER_W)]

    for i in range(W // LANES):
        ridx[pl.ds(i * LANES, LANES)] = lane + i * LANES

    def mk(t):
        k, ph, h = t // 4, (t // 2) % 2, t % 2
        tab = vt_hbm if ph else ut_hbm
        return pltpu.make_async_copy(
            tab.at[:, pl.ds(h * VH, VH)].at[fs[k]],
            (vec0 if t % 2 == 0 else vec1).at[pl.ds(0, VH)], sem)

    mk(0).start()
    mk(1).start()

    @pl.when(s == 0)
    def _():
        pltpu.sync_copy(wu_hbm, sidx_u)
        pltpu.sync_copy(wv_hbm, sidx_v)

    for r in range(RPW):
        for q in range(W // LANES):
            sbuf[r, pl.ds(q * LANES, LANES)] = zero16
    my_rows = pl.ds(s * RPW, RPW)
    pltpu.sync_copy(sbuf, shared.at[my_rows])
    plsc.subcore_barrier()

    def scan(ph, h, ch, buf):
        lo = h * VH
        hi = VH if h == 0 else VH + TAIL

        @plsc.parallel_loop(0, IROWS, unroll=1)
        def body(r2):
            r = ch * IROWS + r2
            for qq in range(W // LANES):
                sl = pl.ds(r2 * W + qq * LANES, LANES)
                csl = pl.ds(qq * LANES, LANES)
                li = idxb[sl] - lo
                m = (li >= 0) & (li < hi)
                g = plsc.load_gather(buf, [jnp.where(m, li, 0)])
                if ph == 0:
                    if h == 0:
                        g1[r, csl] = jnp.where(m, g, 0.0)
                    else:
                        g1[r, csl] = g1[r, csl] + jnp.where(m, g, 0.0)
                else:
                    g1[r, csl] = g1[r, csl] * jnp.where(m, g, 1.0)

    for t in range(NSTREAM):
        k, ph, h = t // 4, (t // 2) % 2, t % 2
        mk(t).wait()
        if h == 1:
            ttab = tv_hbm if ph else tu_hbm
            buf = vec0 if t % 2 == 0 else vec1
            pltpu.sync_copy(ttab.at[fs[k]], tailb)
            for i in range(TAIL // LANES):
                buf[pl.ds(VH + i * LANES, LANES)] = tailb[pl.ds(i * LANES,
                                                               LANES)]
        idx_sp = sidx_v if ph else sidx_u
        for ch in range(BATCH // ICH):
            pltpu.sync_copy(idx_sp.at[pl.ds(ch * ICH, ICH)], idxb)
            scan(ph, h, ch, vec0 if t % 2 == 0 else vec1)
        if t + 2 < NSTREAM:
            mk(t + 2).start()
        if t % 4 == 3:
            pltpu.sync_copy(g1, shared.at[ridx], add=True)

    plsc.subcore_barrier()

    pltpu.sync_copy(shared.at[my_rows], sbuf)

    def add_bias(bias_hbm, widx_hbm):
        half_rows = RPW // 2
        for hh in range(2):
            base = (s * RPW + hh * half_rows) * W
            pltpu.sync_copy(widx_hbm.at[pl.ds(base, half_rows * W)], bidx)
            copies = [pltpu.make_async_copy(
                bias_hbm.at[bidx.at[pl.ds(j * W, W)]],
                bbuf.at[pl.ds(j * W, W)], sem2)
                for j in range(half_rows)]
            for cp in copies:
                cp.start()
            for cp in copies:
                cp.wait()
            for r in range(half_rows):
                for q in range(W // LANES):
                    sl = pl.ds(q * LANES, LANES)
                    sbuf[hh * half_rows + r, sl] = (
                        sbuf[hh * half_rows + r, sl]
                        + bbuf[pl.ds(r * W + q * LANES, LANES)])

    @pl.when(c == 0)
    def _():
        add_bias(ub_hbm, wu_hbm)
        pltpu.sync_copy(sbuf, out0_hbm.at[my_rows])

    @pl.when(c == 1)
    def _():
        add_bias(vb_hbm, wv_hbm)
        pltpu.sync_copy(sbuf, out1_hbm.at[my_rows])


def _glove_sc(wu, wv, ut, ub1, vt, vb1, tu, tv):
    mesh = plsc.VectorSubcoreMesh(core_axis_name="c", subcore_axis_name="s")
    f = pl.kernel(
        _glove_body,
        out_type=(jax.ShapeDtypeStruct((ROWS, W), jnp.float32),
                  jax.ShapeDtypeStruct((ROWS, W), jnp.float32)),
        mesh=mesh,
        scratch_types=[
            pltpu.VMEM((VH + TAIL,), jnp.float32),
            pltpu.VMEM((VH + TAIL,), jnp.float32),
            pltpu.VMEM((ROWS, W), jnp.float32),
            pltpu.VMEM((ICH,), jnp.int32),
            pltpu.VMEM((TAIL,), jnp.float32),
            pltpu.VMEM((RPW, W), jnp.float32),
            pltpu.VMEM((RPW * W // 2,), jnp.float32),
            pltpu.VMEM((RPW * W // 2,), jnp.int32),
            pltpu.VMEM((W,), jnp.int32),
            pltpu.VMEM_SHARED((ROWS, W), jnp.float32),
            pltpu.VMEM_SHARED((BATCH,), jnp.int32),
            pltpu.VMEM_SHARED((BATCH,), jnp.int32),
            pltpu.SemaphoreType.DMA,
            pltpu.SemaphoreType.DMA,
        ],
        compiler_params=pltpu.CompilerParams(needs_layout_passes=False),
    )
    return f(wu, wv, ut, ub1, vt, vb1, tu, tv)


def kernel(word_u, word_v, in_embed, in_bias, out_embed, out_bias):
    wu = word_u.astype(jnp.int32)
    wv = word_v.astype(jnp.int32)
    ut = in_embed.T
    vt = out_embed.T
    out0, out1 = _glove_sc(wu, wv, ut, in_bias.reshape(VOCAB),
                           vt, out_bias.reshape(VOCAB),
                           ut[:, TAIL0:], vt[:, TAIL0:])
    return (out0 + out1).reshape(BATCH)
